# baseline (speedup 1.0000x reference)
import functools

import jax
import jax.numpy as jnp
from jax import lax
from jax.experimental import pallas as pl
from jax.experimental.pallas import tpu as pltpu
from jax.experimental.pallas import tpu_sc as plsc

D = 128
B = 16384
NC = 2
NS = 16
NW = NC * NS
BPW = B // NW
CHUNK = 128
NCHUNK = BPW // CHUNK

_mesh = plsc.VectorSubcoreMesh(core_axis_name="c", subcore_axis_name="s",
                               num_cores=NC, num_subcores=NS)


@functools.partial(
    pl.kernel,
    out_type=(jax.ShapeDtypeStruct((B, D), jnp.float32),
              jax.ShapeDtypeStruct((B, D), jnp.float32)),
    mesh=_mesh,
    scratch_types=[
        pltpu.VMEM((NCHUNK, CHUNK), jnp.int32),
        pltpu.VMEM((CHUNK, D), jnp.float32),
        pltpu.VMEM((CHUNK, D), jnp.float32),
        pltpu.VMEM((CHUNK, D), jnp.float32),
        pltpu.VMEM((CHUNK, D), jnp.float32),
        pltpu.SemaphoreType.DMA,
        pltpu.SemaphoreType.DMA,
        pltpu.SemaphoreType.DMA,
        pltpu.SemaphoreType.DMA,
    ],
)
def _gather2(ids_hbm, shape_hbm, tex_hbm, out_s_hbm, out_t_hbm,
             idx_v, bs0, bs1, bt0, bt1, sg0, sg1, sw0, sw1):
    wid = lax.axis_index("s") * NC + lax.axis_index("c")
    base = wid * BPW
    pltpu.sync_copy(ids_hbm.at[pl.ds(wid * NCHUNK, NCHUNK)], idx_v)
    bufs_s, bufs_t = (bs0, bs1), (bt0, bt1)
    sg, sw = (sg0, sg1), (sw0, sw1)
    g = [None] * NCHUNK
    w = [None] * NCHUNK

    def fire_gather(j):
        p = j % 2
        g[j] = (pltpu.async_copy(shape_hbm.at[idx_v.at[j]], bufs_s[p], sg[p]),
                pltpu.async_copy(tex_hbm.at[idx_v.at[j]], bufs_t[p], sg[p]))

    fire_gather(0)
    for j in range(NCHUNK):
        p = j % 2
        if j + 1 < NCHUNK:
            if j >= 1:
                for cp in w[j - 1]:
                    cp.wait()
            fire_gather(j + 1)
        for cp in g[j]:
            cp.wait()
        dst = pl.ds(base + j * CHUNK, CHUNK)
        w[j] = (pltpu.async_copy(bufs_s[p], out_s_hbm.at[dst], sw[p]),
                pltpu.async_copy(bufs_t[p], out_t_hbm.at[dst], sw[p]))
    for cp in w[NCHUNK - 2] + w[NCHUNK - 1]:
        cp.wait()


def kernel(object_ids, shape_table, texture_table):
    ids2d = object_ids.astype(jnp.int32).reshape(NW * NCHUNK, CHUNK)
    return _gather2(ids2d, shape_table, texture_table)


# speedup vs baseline: 1.0409x; 1.0409x over previous
import functools

import jax
import jax.numpy as jnp
from jax import lax
from jax.experimental import pallas as pl
from jax.experimental.pallas import tpu as pltpu
from jax.experimental.pallas import tpu_sc as plsc

D = 128
B = 16384
NC = 2
NS = 16
NW = NC * NS
BPW = B // NW
CHUNK = 128
NCHUNK = BPW // CHUNK

_mesh = plsc.VectorSubcoreMesh(core_axis_name="c", subcore_axis_name="s",
                               num_cores=NC, num_subcores=NS)


@functools.partial(
    pl.kernel,
    out_type=(jax.ShapeDtypeStruct((B, D), jnp.float32),
              jax.ShapeDtypeStruct((B, D), jnp.float32)),
    mesh=_mesh,
    scratch_types=[
        pltpu.VMEM((NCHUNK, CHUNK), jnp.int32),
        pltpu.VMEM((CHUNK, D), jnp.float32),
        pltpu.VMEM((CHUNK, D), jnp.float32),
        pltpu.VMEM((CHUNK, D), jnp.float32),
        pltpu.VMEM((CHUNK, D), jnp.float32),
        pltpu.VMEM((CHUNK, D), jnp.float32),
        pltpu.VMEM((CHUNK, D), jnp.float32),
        pltpu.SemaphoreType.DMA,
        pltpu.SemaphoreType.DMA,
        pltpu.SemaphoreType.DMA,
        pltpu.SemaphoreType.DMA,
        pltpu.SemaphoreType.DMA,
        pltpu.SemaphoreType.DMA,
    ],
)
def _gather2(ids_hbm, shape_hbm, tex_hbm, out_s_hbm, out_t_hbm,
             idx_v, bs0, bs1, bs2, bt0, bt1, bt2, sg0, sg1, sg2,
             sw0, sw1, sw2):
    wid = lax.axis_index("s") * NC + lax.axis_index("c")
    base = wid * BPW
    pltpu.sync_copy(ids_hbm.at[pl.ds(wid * NCHUNK, NCHUNK)], idx_v)
    bufs_s, bufs_t = (bs0, bs1, bs2), (bt0, bt1, bt2)
    sg, sw = (sg0, sg1, sg2), (sw0, sw1, sw2)
    NBUF = 3
    g = [None] * NCHUNK
    w = [None] * NCHUNK

    def fire_gather(j):
        p = j % NBUF
        g[j] = (pltpu.async_copy(shape_hbm.at[idx_v.at[j]], bufs_s[p], sg[p]),
                pltpu.async_copy(tex_hbm.at[idx_v.at[j]], bufs_t[p], sg[p]))

    waited = [False] * NCHUNK
    for j in range(min(NBUF, NCHUNK)):
        fire_gather(j)
    for j in range(NCHUNK):
        p = j % NBUF
        nxt = j + NBUF - 1
        if j >= 1 and nxt < NCHUNK:
            for cp in w[j - 1]:
                cp.wait()
            waited[j - 1] = True
            fire_gather(nxt)
        for cp in g[j]:
            cp.wait()
        dst = pl.ds(base + j * CHUNK, CHUNK)
        w[j] = (pltpu.async_copy(bufs_s[p], out_s_hbm.at[dst], sw[p]),
                pltpu.async_copy(bufs_t[p], out_t_hbm.at[dst], sw[p]))
    for j in range(NCHUNK):
        if not waited[j]:
            for cp in w[j]:
                cp.wait()


def kernel(object_ids, shape_table, texture_table):
    ids2d = object_ids.astype(jnp.int32).reshape(NW * NCHUNK, CHUNK)
    return _gather2(ids2d, shape_table, texture_table)


# speedup vs baseline: 1.0446x; 1.0036x over previous
import functools

import jax
import jax.numpy as jnp
from jax import lax
from jax.experimental import pallas as pl
from jax.experimental.pallas import tpu as pltpu
from jax.experimental.pallas import tpu_sc as plsc

D = 128
B = 16384
NC = 2
NS = 16
NW = NC * NS
BPW = B // NW
CHUNK = 64
NCHUNK = BPW // CHUNK
NBUF = 6

_mesh = plsc.VectorSubcoreMesh(core_axis_name="c", subcore_axis_name="s",
                               num_cores=NC, num_subcores=NS)

_scratch = (
    [pltpu.VMEM((NCHUNK, CHUNK), jnp.int32)]
    + [pltpu.VMEM((CHUNK, D), jnp.float32)] * NBUF
    + [pltpu.VMEM((CHUNK, D), jnp.float32)] * NBUF
    + [pltpu.SemaphoreType.DMA] * NBUF
    + [pltpu.SemaphoreType.DMA] * NBUF
)


@functools.partial(
    pl.kernel,
    out_type=(jax.ShapeDtypeStruct((B, D), jnp.float32),
              jax.ShapeDtypeStruct((B, D), jnp.float32)),
    mesh=_mesh,
    scratch_types=_scratch,
)
def _gather2(ids_hbm, shape_hbm, tex_hbm, out_s_hbm, out_t_hbm,
             idx_v, *scr):
    bufs_s = scr[0:NBUF]
    bufs_t = scr[NBUF:2 * NBUF]
    sg = scr[2 * NBUF:3 * NBUF]
    sw = scr[3 * NBUF:4 * NBUF]
    wid = lax.axis_index("s") * NC + lax.axis_index("c")
    base = wid * BPW
    pltpu.sync_copy(ids_hbm.at[pl.ds(wid * NCHUNK, NCHUNK)], idx_v)
    g = [None] * NCHUNK
    w = [None] * NCHUNK
    waited = [False] * NCHUNK

    def fire_gather(j):
        p = j % NBUF
        g[j] = (pltpu.async_copy(shape_hbm.at[idx_v.at[j]], bufs_s[p], sg[p]),
                pltpu.async_copy(tex_hbm.at[idx_v.at[j]], bufs_t[p], sg[p]))

    for j in range(min(NBUF, NCHUNK)):
        fire_gather(j)
    for j in range(NCHUNK):
        p = j % NBUF
        nxt = j + NBUF - 1
        if j >= 1 and nxt < NCHUNK:
            for cp in w[j - 1]:
                cp.wait()
            waited[j - 1] = True
            fire_gather(nxt)
        for cp in g[j]:
            cp.wait()
        dst = pl.ds(base + j * CHUNK, CHUNK)
        w[j] = (pltpu.async_copy(bufs_s[p], out_s_hbm.at[dst], sw[p]),
                pltpu.async_copy(bufs_t[p], out_t_hbm.at[dst], sw[p]))
    for j in range(NCHUNK):
        if not waited[j]:
            for cp in w[j]:
                cp.wait()


def kernel(object_ids, shape_table, texture_table):
    ids2d = object_ids.astype(jnp.int32).reshape(NW * NCHUNK, CHUNK)
    return _gather2(ids2d, shape_table, texture_table)
